# Initial kernel scaffold; baseline (speedup 1.0000x reference)
import jax
import jax.numpy as jnp
from jax.experimental import pallas as pl


def kernel(embeddings, target):
    raise NotImplementedError("write your pallas kernel here")


# speedup vs baseline: 661.2398x; 661.2398x over previous
import jax
import jax.numpy as jnp
from jax.experimental import pallas as pl

_N = 512
_D = 128
_MARGIN = 1.0
_N_PAIRS = _N * (_N - 1) // 2


def _loss_kernel(emb_ref, tgt_ref, out_ref):
    e = emb_ref[...]
    g = jax.lax.dot_general(
        e, e, (((1,), (1,)), ((), ())),
        preferred_element_type=jnp.float32,
    )
    sq = jnp.sum(e * e, axis=1, keepdims=True)
    dist2 = jnp.maximum(sq + jnp.transpose(sq) - 2.0 * g, 0.0)

    tcol = tgt_ref[...]
    same = tcol == jnp.transpose(tcol)

    row_i = jax.lax.broadcasted_iota(jnp.int32, (_N, _N), 0)
    col_j = jax.lax.broadcasted_iota(jnp.int32, (_N, _N), 1)
    upper = row_i < col_j

    neg = jnp.square(jnp.maximum(_MARGIN - jnp.sqrt(dist2), 0.0))
    contrib = jnp.where(same, dist2, neg)
    contrib = jnp.where(upper, contrib, 0.0)
    out_ref[...] = (jnp.sum(contrib) * (1.0 / _N_PAIRS)).reshape(1, 1)


def kernel(embeddings, target):
    out = pl.pallas_call(
        _loss_kernel,
        out_shape=jax.ShapeDtypeStruct((1, 1), jnp.float32),
    )(embeddings, target.reshape(_N, 1))
    return out[0, 0]
